# baseline (speedup 1.0000x reference)
import jax
import jax.numpy as jnp
from jax.experimental import pallas as pl
from jax.experimental.pallas import tpu as pltpu

_N = 10000
_BI0 = 160
_NP0 = 63
_BI1 = 256
_NP1 = 40
_NQPAD = 10240
_RES = 2560
_RB0 = _RES // _BI0
_RB1 = _RES // _BI1
_SA = 2.0 ** 21
_S2 = 2.0 ** 12
_INV = 2.0 ** -33


def _elu(x):
    return jnp.where(x > 0, x, jnp.exp(x) - 1.0)


def _gcn_kernel(adj_ref, x_ref, w1_ref, b1_ref, w2_ref, b2_ref, fcw_ref,
                fcb_ref, oq_ref, o_ref,
                g1_ref, g2q_ref, res_ref, wstage_ref, rstage_ref,
                wsem, rsem):
    i = pl.program_id(0)

    @pl.when(i == 0)
    def _():
        g1 = jnp.dot(x_ref[...], w1_ref[...],
                     preferred_element_type=jnp.float32)
        g1_ref[...] = g1.astype(jnp.bfloat16)

    @pl.when(i < _NP0)
    def _():
        ab = adj_ref[...]
        q = (ab * _SA).astype(jnp.float8_e4m3fn)
        acc = jnp.dot(ab.astype(jnp.bfloat16), g1_ref[...],
                      preferred_element_type=jnp.float32)
        h = _elu(acc + b1_ref[...])
        g2 = jnp.dot(h, w2_ref[...], preferred_element_type=jnp.float32)
        g2q_ref[pl.ds(i * _BI0, _BI0), :] = (g2 * _S2).astype(jnp.float8_e4m3fn)

        @pl.when(i < _RB0)
        def _():
            res_ref[pl.ds(i * _BI0, _BI0), :] = q

        for s in (0, 1):
            @pl.when((i >= _RB0) & (jax.lax.rem(i, 2) == s))
            def _(s=s):
                @pl.when(i >= _RB0 + 2)
                def _():
                    pltpu.make_async_copy(
                        wstage_ref.at[s],
                        oq_ref.at[pl.ds((i - 2) * _BI0, _BI0), :],
                        wsem.at[s]).wait()
                wstage_ref[s] = q
                pltpu.make_async_copy(
                    wstage_ref.at[s],
                    oq_ref.at[pl.ds(i * _BI0, _BI0), :],
                    wsem.at[s]).start()

    @pl.when(i == _NP0)
    def _():
        pltpu.make_async_copy(
            wstage_ref.at[0], oq_ref.at[pl.ds((_NP0 - 1) * _BI0, _BI0), :],
            wsem.at[0]).wait()
        pltpu.make_async_copy(
            wstage_ref.at[1], oq_ref.at[pl.ds((_NP0 - 2) * _BI0, _BI0), :],
            wsem.at[1]).wait()
        pltpu.make_async_copy(
            oq_ref.at[pl.ds(_RB1 * _BI1, _BI1), :], rstage_ref.at[0],
            rsem.at[0]).start()
        pltpu.make_async_copy(
            oq_ref.at[pl.ds((_RB1 + 1) * _BI1, _BI1), :], rstage_ref.at[1],
            rsem.at[1]).start()

    @pl.when(i >= _NP0)
    def _():
        j = i - _NP0

        def compute(qblk):
            acc = jnp.dot(qblk, g2q_ref[pl.ds(0, _N), :],
                          preferred_element_type=jnp.float32) * _INV
            h = _elu(acc + b2_ref[...])
            o_ref[...] = jnp.dot(h, fcw_ref[...],
                                 preferred_element_type=jnp.float32) + fcb_ref[...]

        @pl.when(j < _RB1)
        def _():
            compute(res_ref[pl.ds(j * _BI1, _BI1), :])

        for s in (0, 1):
            @pl.when((j >= _RB1) & (jax.lax.rem(j, 2) == s))
            def _(s=s):
                pltpu.make_async_copy(
                    oq_ref.at[pl.ds(j * _BI1, _BI1), :], rstage_ref.at[s],
                    rsem.at[s]).wait()
                compute(rstage_ref[s])

                @pl.when(j + 2 < _NP1)
                def _():
                    pltpu.make_async_copy(
                        oq_ref.at[pl.ds((j + 2) * _BI1, _BI1), :],
                        rstage_ref.at[s], rsem.at[s]).start()


@jax.jit
def kernel(input, adj, W1, b1, W2, b2, fc_W, fc_b):
    n, n_in = input.shape
    n_hid = W1.shape[1]
    n_out = fc_W.shape[1]
    grid = (_NP0 + _NP1,)

    p0 = lambda i: (jnp.minimum(i, _NP0 - 1), 0)
    const = lambda i: (0, 0)

    _, out = pl.pallas_call(
        _gcn_kernel,
        grid=grid,
        in_specs=[
            pl.BlockSpec((_BI0, n), p0),
            pl.BlockSpec((n, n_in), const),
            pl.BlockSpec((n_in, n_hid), const),
            pl.BlockSpec((1, n_hid), const),
            pl.BlockSpec((n_hid, n_hid), const),
            pl.BlockSpec((1, n_hid), const),
            pl.BlockSpec((n_hid, n_out), const),
            pl.BlockSpec((1, n_out), const),
        ],
        out_specs=[
            pl.BlockSpec(memory_space=pltpu.MemorySpace.HBM),
            pl.BlockSpec((_BI1, n_out),
                         lambda i: (jnp.maximum(i - _NP0, 0), 0)),
        ],
        out_shape=[
            jax.ShapeDtypeStruct((_NQPAD, n), jnp.float8_e4m3fn),
            jax.ShapeDtypeStruct((n, n_out), jnp.float32),
        ],
        scratch_shapes=[
            pltpu.VMEM((n, n_hid), jnp.bfloat16),
            pltpu.VMEM((_NP0 * _BI0, n_hid), jnp.float8_e4m3fn),
            pltpu.VMEM((_RES, n), jnp.float8_e4m3fn),
            pltpu.VMEM((2, _BI0, n), jnp.float8_e4m3fn),
            pltpu.VMEM((2, _BI1, n), jnp.float8_e4m3fn),
            pltpu.SemaphoreType.DMA((2,)),
            pltpu.SemaphoreType.DMA((2,)),
        ],
        compiler_params=pltpu.CompilerParams(
            dimension_semantics=("arbitrary",),
        ),
    )(adj, input, W1, b1.reshape(1, n_hid), W2, b2.reshape(1, n_hid),
      fc_W, fc_b.reshape(1, n_out))

    return out


# speedup vs baseline: 1.0538x; 1.0538x over previous
import jax
import jax.numpy as jnp
from jax.experimental import pallas as pl
from jax.experimental.pallas import tpu as pltpu

_BI = 400
_BI2 = 2000
_SA = 2.0 ** 21
_S2 = 2.0 ** 12
_INV = 2.0 ** -33


def _elu(x):
    return jnp.where(x > 0, x, jnp.exp(x) - 1.0)


def _layer1_kernel(adj_ref, x_ref, w1_ref, b1_ref, w2_ref,
                   adjq_ref, g2q_ref, g1_ref):
    i = pl.program_id(0)

    @pl.when(i == 0)
    def _():
        g1_ref[...] = jnp.dot(x_ref[...], w1_ref[...],
                              preferred_element_type=jnp.float32)

    ab = adj_ref[...]
    adjq_ref[...] = (ab * _SA).astype(jnp.float8_e4m3fn)
    acc = jnp.dot(ab, g1_ref[...], preferred_element_type=jnp.float32)
    h = _elu(acc + b1_ref[...])
    g2 = jnp.dot(h, w2_ref[...], preferred_element_type=jnp.float32)
    g2q_ref[...] = (g2 * _S2).astype(jnp.float8_e4m3fn)


def _layer2_kernel(adjq_ref, g2q_ref, b2_ref, fcw_ref, fcb_ref, o_ref):
    acc = jnp.dot(adjq_ref[...], g2q_ref[...],
                  preferred_element_type=jnp.float32) * _INV
    h = _elu(acc + b2_ref[...])
    o_ref[...] = jnp.dot(h, fcw_ref[...],
                         preferred_element_type=jnp.float32) + fcb_ref[...]


@jax.jit
def kernel(input, adj, W1, b1, W2, b2, fc_W, fc_b):
    n, n_in = input.shape
    n_hid = W1.shape[1]
    n_out = fc_W.shape[1]

    adj_q, g2_q = pl.pallas_call(
        _layer1_kernel,
        grid=(n // _BI,),
        in_specs=[
            pl.BlockSpec((_BI, n), lambda i: (i, 0)),
            pl.BlockSpec((n, n_in), lambda i: (0, 0)),
            pl.BlockSpec((n_in, n_hid), lambda i: (0, 0)),
            pl.BlockSpec((1, n_hid), lambda i: (0, 0)),
            pl.BlockSpec((n_hid, n_hid), lambda i: (0, 0)),
        ],
        out_specs=[
            pl.BlockSpec((_BI, n), lambda i: (i, 0)),
            pl.BlockSpec((_BI, n_hid), lambda i: (i, 0)),
        ],
        out_shape=[
            jax.ShapeDtypeStruct((n, n), jnp.float8_e4m3fn),
            jax.ShapeDtypeStruct((n, n_hid), jnp.float8_e4m3fn),
        ],
        scratch_shapes=[pltpu.VMEM((n, n_hid), jnp.float32)],
        compiler_params=pltpu.CompilerParams(
            dimension_semantics=("arbitrary",),
        ),
    )(adj, input, W1, b1.reshape(1, n_hid), W2)

    out = pl.pallas_call(
        _layer2_kernel,
        grid=(n // _BI2,),
        in_specs=[
            pl.BlockSpec((_BI2, n), lambda i: (i, 0)),
            pl.BlockSpec((n, n_hid), lambda i: (0, 0)),
            pl.BlockSpec((1, n_hid), lambda i: (0, 0)),
            pl.BlockSpec((n_hid, n_out), lambda i: (0, 0)),
            pl.BlockSpec((1, n_out), lambda i: (0, 0)),
        ],
        out_specs=pl.BlockSpec((_BI2, n_out), lambda i: (i, 0)),
        out_shape=jax.ShapeDtypeStruct((n, n_out), jnp.float32),
        compiler_params=pltpu.CompilerParams(
            dimension_semantics=("arbitrary",),
        ),
    )(adj_q, g2_q, b2.reshape(1, n_hid), fc_W, fc_b.reshape(1, n_out))

    return out


# speedup vs baseline: 1.0550x; 1.0012x over previous
import jax
import jax.numpy as jnp
from jax.experimental import pallas as pl
from jax.experimental.pallas import tpu as pltpu

_BI = 200
_BI2 = 1000
_SA = 2.0 ** 21
_S2 = 2.0 ** 12
_INV = 2.0 ** -33


def _elu(x):
    return jnp.where(x > 0, x, jnp.exp(x) - 1.0)


def _layer1_kernel(adj_ref, x_ref, w1_ref, b1_ref, w2_ref,
                   adjq_ref, g2q_ref, g1_ref):
    i = pl.program_id(0)

    @pl.when(i == 0)
    def _():
        g1_ref[...] = jnp.dot(x_ref[...], w1_ref[...],
                              preferred_element_type=jnp.float32)

    ab = adj_ref[...]
    adjq_ref[...] = (ab * _SA).astype(jnp.float8_e4m3fn)
    acc = jnp.dot(ab, g1_ref[...], preferred_element_type=jnp.float32)
    h = _elu(acc + b1_ref[...])
    g2 = jnp.dot(h, w2_ref[...], preferred_element_type=jnp.float32)
    g2q_ref[...] = (g2 * _S2).astype(jnp.float8_e4m3fn)


def _layer2_kernel(adjq_ref, g2q_ref, b2_ref, fcw_ref, fcb_ref, o_ref):
    acc = jnp.dot(adjq_ref[...], g2q_ref[...],
                  preferred_element_type=jnp.float32) * _INV
    h = _elu(acc + b2_ref[...])
    o_ref[...] = jnp.dot(h, fcw_ref[...],
                         preferred_element_type=jnp.float32) + fcb_ref[...]


@jax.jit
def kernel(input, adj, W1, b1, W2, b2, fc_W, fc_b):
    n, n_in = input.shape
    n_hid = W1.shape[1]
    n_out = fc_W.shape[1]

    adj_q, g2_q = pl.pallas_call(
        _layer1_kernel,
        grid=(n // _BI,),
        in_specs=[
            pl.BlockSpec((_BI, n), lambda i: (i, 0)),
            pl.BlockSpec((n, n_in), lambda i: (0, 0)),
            pl.BlockSpec((n_in, n_hid), lambda i: (0, 0)),
            pl.BlockSpec((1, n_hid), lambda i: (0, 0)),
            pl.BlockSpec((n_hid, n_hid), lambda i: (0, 0)),
        ],
        out_specs=[
            pl.BlockSpec((_BI, n), lambda i: (i, 0)),
            pl.BlockSpec((_BI, n_hid), lambda i: (i, 0)),
        ],
        out_shape=[
            jax.ShapeDtypeStruct((n, n), jnp.float8_e4m3fn),
            jax.ShapeDtypeStruct((n, n_hid), jnp.float8_e4m3fn),
        ],
        scratch_shapes=[pltpu.VMEM((n, n_hid), jnp.float32)],
        compiler_params=pltpu.CompilerParams(
            dimension_semantics=("arbitrary",),
        ),
    )(adj, input, W1, b1.reshape(1, n_hid), W2)

    out = pl.pallas_call(
        _layer2_kernel,
        grid=(n // _BI2,),
        in_specs=[
            pl.BlockSpec((_BI2, n), lambda i: (i, 0)),
            pl.BlockSpec((n, n_hid), lambda i: (0, 0)),
            pl.BlockSpec((1, n_hid), lambda i: (0, 0)),
            pl.BlockSpec((n_hid, n_out), lambda i: (0, 0)),
            pl.BlockSpec((1, n_out), lambda i: (0, 0)),
        ],
        out_specs=pl.BlockSpec((_BI2, n_out), lambda i: (i, 0)),
        out_shape=jax.ShapeDtypeStruct((n, n_out), jnp.float32),
        compiler_params=pltpu.CompilerParams(
            dimension_semantics=("arbitrary",),
        ),
    )(adj_q, g2_q, b2.reshape(1, n_hid), fc_W, fc_b.reshape(1, n_out))

    return out


# speedup vs baseline: 1.0677x; 1.0120x over previous
import jax
import jax.numpy as jnp
from jax.experimental import pallas as pl
from jax.experimental.pallas import tpu as pltpu

_BI = 400
_BI2 = 1000
_SA = 2.0 ** 21
_S2 = 2.0 ** 12
_INV = 2.0 ** -33


def _elu(x):
    return jnp.where(x > 0, x, jnp.exp(x) - 1.0)


def _layer1_kernel(adj_ref, x_ref, w1_ref, b1_ref, w2_ref,
                   adjq_ref, g2q_ref, g1_ref):
    i = pl.program_id(0)

    @pl.when(i == 0)
    def _():
        g1_ref[...] = jnp.dot(x_ref[...], w1_ref[...],
                              preferred_element_type=jnp.float32)

    ab = adj_ref[...]
    adjq_ref[...] = (ab * _SA).astype(jnp.float8_e4m3fn)
    acc = jnp.dot(ab, g1_ref[...], preferred_element_type=jnp.float32)
    h = _elu(acc + b1_ref[...])
    g2 = jnp.dot(h, w2_ref[...], preferred_element_type=jnp.float32)
    g2q_ref[...] = (g2 * _S2).astype(jnp.float8_e4m3fn)


def _layer2_kernel(adjq_ref, g2q_ref, b2_ref, fcw_ref, fcb_ref, o_ref):
    acc = jnp.dot(adjq_ref[...], g2q_ref[...],
                  preferred_element_type=jnp.float32) * _INV
    h = _elu(acc + b2_ref[...])
    o_ref[...] = jnp.dot(h, fcw_ref[...],
                         preferred_element_type=jnp.float32) + fcb_ref[...]


@jax.jit
def kernel(input, adj, W1, b1, W2, b2, fc_W, fc_b):
    n, n_in = input.shape
    n_hid = W1.shape[1]
    n_out = fc_W.shape[1]

    adj_q, g2_q = pl.pallas_call(
        _layer1_kernel,
        grid=(n // _BI,),
        in_specs=[
            pl.BlockSpec((_BI, n), lambda i: (i, 0)),
            pl.BlockSpec((n, n_in), lambda i: (0, 0)),
            pl.BlockSpec((n_in, n_hid), lambda i: (0, 0)),
            pl.BlockSpec((1, n_hid), lambda i: (0, 0)),
            pl.BlockSpec((n_hid, n_hid), lambda i: (0, 0)),
        ],
        out_specs=[
            pl.BlockSpec((_BI, n), lambda i: (i, 0)),
            pl.BlockSpec((_BI, n_hid), lambda i: (i, 0)),
        ],
        out_shape=[
            jax.ShapeDtypeStruct((n, n), jnp.float8_e4m3fn),
            jax.ShapeDtypeStruct((n, n_hid), jnp.float8_e4m3fn),
        ],
        scratch_shapes=[pltpu.VMEM((n, n_hid), jnp.float32)],
        compiler_params=pltpu.CompilerParams(
            dimension_semantics=("arbitrary",),
        ),
    )(adj, input, W1, b1.reshape(1, n_hid), W2)

    out = pl.pallas_call(
        _layer2_kernel,
        grid=(n // _BI2,),
        in_specs=[
            pl.BlockSpec((_BI2, n), lambda i: (i, 0)),
            pl.BlockSpec((n, n_hid), lambda i: (0, 0)),
            pl.BlockSpec((1, n_hid), lambda i: (0, 0)),
            pl.BlockSpec((n_hid, n_out), lambda i: (0, 0)),
            pl.BlockSpec((1, n_out), lambda i: (0, 0)),
        ],
        out_specs=pl.BlockSpec((_BI2, n_out), lambda i: (i, 0)),
        out_shape=jax.ShapeDtypeStruct((n, n_out), jnp.float32),
        compiler_params=pltpu.CompilerParams(
            dimension_semantics=("arbitrary",),
        ),
    )(adj_q, g2_q, b2.reshape(1, n_hid), fc_W, fc_b.reshape(1, n_out))

    return out
